# baseline (speedup 1.0000x reference)
import jax
import jax.numpy as jnp
from jax import lax
from jax.experimental import pallas as pl
from jax.experimental.pallas import tpu as pltpu
from jax.experimental.pallas import tpu_sc as plsc

N = 10000
E = 320000
D = 128
NG = 16
NCLS = 10

SC_CORES = 2
SC_TILES = 16
NW = SC_CORES * SC_TILES
EPW = E // NW
CH = 80
NCHUNK = EPW // CH
ROWS_T = 640
ROWS_LAST = N - (SC_TILES - 1) * ROWS_T

_MESH = plsc.VectorSubcoreMesh(
    core_axis_name="c", subcore_axis_name="s",
    num_cores=SC_CORES, num_subcores=SC_TILES)


def _zero_rows_buf(rows, width):
    z = jnp.zeros((16,), jnp.float32)

    def zr(j, c):
        for k in range(width // 16):
            rows[j, pl.ds(16 * k, 16)] = z
        return c
    lax.fori_loop(0, CH, zr, 0)


def _zero_acc(rows, acc, sid, width):
    _zero_rows_buf(rows, width)
    zsl = rows.at[pl.ds(0, CH)]
    start = pl.multiple_of(sid * ROWS_T, 16)

    @pl.when(sid < SC_TILES - 1)
    def _():
        for k in range(ROWS_T // CH):
            pltpu.sync_copy(zsl, acc.at[pl.ds(start + CH * k, CH)])

    @pl.when(sid == SC_TILES - 1)
    def _():
        for k in range(ROWS_LAST // CH):
            pltpu.sync_copy(zsl, acc.at[pl.ds(start + CH * k, CH)])


def _writeback(acc, out_hbm, cid, sid):
    start = pl.multiple_of(sid * ROWS_T, 16)

    @pl.when(sid < SC_TILES - 1)
    def _():
        pltpu.sync_copy(acc.at[pl.ds(start, ROWS_T)],
                        out_hbm.at[cid, pl.ds(start, ROWS_T)])

    @pl.when(sid == SC_TILES - 1)
    def _():
        pltpu.sync_copy(acc.at[pl.ds(start, ROWS_LAST)],
                        out_hbm.at[cid, pl.ds(start, ROWS_LAST)])


NBUF = 4
ROUNDS = (NCHUNK - 1) // NBUF


def _scale_rows(rows, wv, b, unroll=2):
    @plsc.parallel_loop(0, CH // 16, 1, unroll=unroll)
    def grp(g):
        wvec = wv[b, pl.ds(16 * g, 16)]
        for k in range(16):
            wb = jnp.full((16,), wvec[k], jnp.float32)
            j = b * CH + 16 * g + k
            for m in range(D // 16):
                sl = pl.ds(16 * m, 16)
                rows[j, sl] = rows[j, sl] * wb


def _fill_rows(rows, wv, b, unroll=2):
    @plsc.parallel_loop(0, CH // 16, 1, unroll=unroll)
    def grp(g):
        wvec = wv[b, pl.ds(16 * g, 16)]
        for k in range(16):
            wb = jnp.full((16,), wvec[k], jnp.float32)
            j = b * CH + 16 * g + k
            for m in range(D // 16):
                rows[j, pl.ds(16 * m, 16)] = wb


def _deg_body(col_hbm, w_hbm, out_hbm, acc, colv, wv, rows, stg_sem, sc_sem):
    cid = lax.axis_index("c")
    sid = lax.axis_index("s")
    wid = cid * SC_TILES + sid
    base = pl.multiple_of(wid * EPW, 16)

    def stage(g, b):
        off = pl.multiple_of(base + (g * NBUF + b) * CH, 16)
        pltpu.async_copy(col_hbm.at[pl.ds(off, CH)], colv.at[b], stg_sem.at[b])
        pltpu.async_copy(w_hbm.at[pl.ds(off, CH)], wv.at[b], stg_sem.at[b])

    for b in range(NBUF):
        stage(0, b)
    _zero_acc(rows, acc, sid, D)
    plsc.subcore_barrier()

    def rnd(g, carry):
        for b in range(NBUF):
            rsl = rows.at[pl.ds(b * CH, CH)]
            pltpu.make_async_copy(col_hbm.at[pl.ds(0, CH)], colv.at[b],
                                  stg_sem.at[b]).wait()
            pltpu.make_async_copy(w_hbm.at[pl.ds(0, CH)], wv.at[b],
                                  stg_sem.at[b]).wait()
            _fill_rows(rows, wv, b)
            pltpu.async_copy(rsl, acc.at[colv.at[b]], sc_sem.at[b], add=True)
        for b in range(NBUF):
            pltpu.make_async_copy(rows.at[pl.ds(b * CH, CH)],
                                  acc.at[colv.at[b]], sc_sem.at[b]).wait()

            @pl.when(g < ROUNDS - 1)
            def _():
                stage(g + 1, b)
        return carry
    lax.fori_loop(0, ROUNDS, rnd, 0)
    off = pl.multiple_of(base + (NCHUNK - 1) * CH, 16)
    rsl0 = rows.at[pl.ds(0, CH)]
    pltpu.sync_copy(col_hbm.at[pl.ds(off, CH)], colv.at[0])
    pltpu.sync_copy(w_hbm.at[pl.ds(off, CH)], wv.at[0])
    _fill_rows(rows, wv, 0)
    pltpu.sync_copy(rsl0, acc.at[colv.at[0]], add=True)
    plsc.subcore_barrier()
    _writeback(acc, out_hbm, cid, sid)


def _spmm_body(h_hbm, row_hbm, col_hbm, w_hbm, out_hbm,
               acc, rowv, colv, wv, rows, stg_sem, g_sem, sc_sem):
    cid = lax.axis_index("c")
    sid = lax.axis_index("s")
    wid = cid * SC_TILES + sid
    base = pl.multiple_of(wid * EPW, 16)

    def stage(g, b):
        off = pl.multiple_of(base + (g * NBUF + b) * CH, 16)
        pltpu.async_copy(row_hbm.at[pl.ds(off, CH)], rowv.at[b], stg_sem.at[b])
        pltpu.async_copy(col_hbm.at[pl.ds(off, CH)], colv.at[b], stg_sem.at[b])
        pltpu.async_copy(w_hbm.at[pl.ds(off, CH)], wv.at[b], stg_sem.at[b])

    for b in range(NBUF):
        stage(0, b)
    _zero_acc(rows, acc, sid, D)
    plsc.subcore_barrier()

    def rnd(g, carry):
        for b in range(NBUF):
            rsl = rows.at[pl.ds(b * CH, CH)]
            pltpu.make_async_copy(row_hbm.at[pl.ds(0, CH)], rowv.at[b],
                                  stg_sem.at[b]).wait()
            pltpu.make_async_copy(col_hbm.at[pl.ds(0, CH)], colv.at[b],
                                  stg_sem.at[b]).wait()
            pltpu.make_async_copy(w_hbm.at[pl.ds(0, CH)], wv.at[b],
                                  stg_sem.at[b]).wait()
            pltpu.async_copy(h_hbm.at[rowv.at[b]], rsl, g_sem.at[b])
        for b in range(NBUF):
            rsl = rows.at[pl.ds(b * CH, CH)]
            pltpu.make_async_copy(h_hbm.at[rowv.at[b]], rsl,
                                  g_sem.at[b]).wait()
            _scale_rows(rows, wv, b)
            pltpu.async_copy(rsl, acc.at[colv.at[b]], sc_sem.at[b], add=True)
        for b in range(NBUF):
            pltpu.make_async_copy(rows.at[pl.ds(b * CH, CH)],
                                  acc.at[colv.at[b]], sc_sem.at[b]).wait()

            @pl.when(g < ROUNDS - 1)
            def _():
                stage(g + 1, b)
        return carry
    lax.fori_loop(0, ROUNDS, rnd, 0)
    off = pl.multiple_of(base + (NCHUNK - 1) * CH, 16)
    rsl0 = rows.at[pl.ds(0, CH)]
    pltpu.sync_copy(row_hbm.at[pl.ds(off, CH)], rowv.at[0])
    pltpu.sync_copy(col_hbm.at[pl.ds(off, CH)], colv.at[0])
    pltpu.sync_copy(w_hbm.at[pl.ds(off, CH)], wv.at[0])
    pltpu.async_copy(h_hbm.at[rowv.at[0]], rsl0, g_sem.at[0]).wait()
    _scale_rows(rows, wv, 0)
    pltpu.sync_copy(rsl0, acc.at[colv.at[0]], add=True)
    plsc.subcore_barrier()
    _writeback(acc, out_hbm, cid, sid)


_deg_call = pl.kernel(
    _deg_body,
    out_type=jax.ShapeDtypeStruct((SC_CORES, N, D), jnp.float32),
    mesh=_MESH,
    scratch_types=[
        pltpu.VMEM_SHARED((N, D), jnp.float32),
        pltpu.VMEM((NBUF, CH), jnp.int32),
        pltpu.VMEM((NBUF, CH), jnp.float32),
        pltpu.VMEM((NBUF * CH, D), jnp.float32),
        pltpu.SemaphoreType.DMA((NBUF,)),
        pltpu.SemaphoreType.DMA((NBUF,)),
    ],
)

_spmm_call = pl.kernel(
    _spmm_body,
    out_type=jax.ShapeDtypeStruct((SC_CORES, N, D), jnp.float32),
    mesh=_MESH,
    scratch_types=[
        pltpu.VMEM_SHARED((N, D), jnp.float32),
        pltpu.VMEM((NBUF, CH), jnp.int32),
        pltpu.VMEM((NBUF, CH), jnp.int32),
        pltpu.VMEM((NBUF, CH), jnp.float32),
        pltpu.VMEM((NBUF * CH, D), jnp.float32),
        pltpu.SemaphoreType.DMA((NBUF,)),
        pltpu.SemaphoreType.DMA((NBUF,)),
        pltpu.SemaphoreType.DMA((NBUF,)),
    ],
)


def _relu(x):
    return jnp.maximum(x, 0.0)


def _dis_body(degp, dis):
    deg = degp[0, :, 0:1] + degp[1, :, 0:1]
    dis1 = jnp.where(deg > 0, lax.rsqrt(jnp.maximum(deg, 1e-12)), 0.0)
    dis[...] = jnp.broadcast_to(dis1, (N, D))


def _enc_body(x1, x2, We1a, be1a, We1b, be1b, We2a, be2a, We2b, be2b,
              Wg1, DIS, t1, u, h2o):
    f32 = jnp.float32
    h1 = _relu(jnp.dot(x1[...], We1a[...], preferred_element_type=f32)
               + be1a[...])
    h1 = jnp.dot(h1, We1b[...], preferred_element_type=f32) + be1b[...]
    h2 = _relu(jnp.dot(x2[...], We2a[...], preferred_element_type=f32)
               + be2a[...])
    h2 = jnp.dot(h2, We2b[...], preferred_element_type=f32) + be2b[...]
    t1[...] = DIS[...] * jnp.dot(h1, Wg1[...], preferred_element_type=f32)
    u[...] = DIS[...] * h2
    h2o[...] = h2


def _mid1_body(Pa, Pb, DIS, bg1, Wg2, h2, Wc1_0, Wc1_1, t2, v, c1acc):
    f32 = jnp.float32
    g = _relu(DIS[...] * (Pa[0] + Pa[1]) + bg1[...])
    t2[...] = DIS[...] * jnp.dot(g, Wg2[...], preferred_element_type=f32)
    Tx1 = -(DIS[...] * (Pb[0] + Pb[1]))
    v[...] = DIS[...] * Tx1
    c1acc[...] = (jnp.dot(h2[...], Wc1_0[...], preferred_element_type=f32)
                  + jnp.dot(Tx1, Wc1_1[...], preferred_element_type=f32))


def _mid2_body(Pc, Pd, DIS, bg2, h2, c1acc, Wc1_2, bc1, g2, c1, w1):
    f32 = jnp.float32
    g2[...] = _relu(DIS[...] * (Pc[0] + Pc[1]) + bg2[...])
    Tx2 = -2.0 * (DIS[...] * (Pd[0] + Pd[1])) - h2[...]
    c1v = _relu(c1acc[...]
                + jnp.dot(Tx2, Wc1_2[...], preferred_element_type=f32)
                + bc1[...])
    c1[...] = c1v
    w1[...] = DIS[...] * c1v


def _mid3_body(Pe, DIS, c1, Wc2_0, Wc2_1, vb, c2acc):
    f32 = jnp.float32
    Tx1b = -(DIS[...] * (Pe[0] + Pe[1]))
    vb[...] = DIS[...] * Tx1b
    c2acc[...] = (jnp.dot(c1[...], Wc2_0[...], preferred_element_type=f32)
                  + jnp.dot(Tx1b, Wc2_1[...], preferred_element_type=f32))


def _final_body(Pf, DIS, c1, c2acc, Wc2_2, bc2, g2, WhfT, WhfB, bhf,
                batch, Wh1, bh1, Wh2, bh2, logits):
    f32 = jnp.float32
    Tx2b = -2.0 * (DIS[...] * (Pf[0] + Pf[1])) - c1[...]
    c2 = _relu(c2acc[...]
               + jnp.dot(Tx2b, Wc2_2[...], preferred_element_type=f32)
               + bc2[...])
    fh = _relu(jnp.dot(g2[...], WhfT[...], preferred_element_type=f32)
               + jnp.dot(c2, WhfB[...], preferred_element_type=f32)
               + bhf[...])
    gid = lax.broadcasted_iota(jnp.int32, (NG, N), 0)
    oh = (gid == jnp.broadcast_to(batch[...], (NG, N))).astype(f32)
    sums = jnp.dot(oh, fh, preferred_element_type=f32)
    cnt = jnp.sum(oh, axis=1, keepdims=True)
    pooled = sums / jnp.maximum(cnt, 1.0)
    hh = _relu(jnp.dot(pooled, Wh1[...], preferred_element_type=f32)
               + bh1[...])
    logits[...] = jnp.dot(hh, Wh2[...], preferred_element_type=f32) + bh2[...]


def _tc(body, *outs):
    return pl.pallas_call(body, out_shape=[jax.ShapeDtypeStruct(s, jnp.float32)
                                           for s in outs])


_dis_call = _tc(_dis_body, (N, D))
_enc_call = _tc(_enc_body, (N, D), (N, D), (N, D))
_mid1_call = _tc(_mid1_body, (N, D), (N, D), (N, D))
_mid2_call = _tc(_mid2_body, (N, D), (N, D), (N, D))
_mid3_call = _tc(_mid3_body, (N, D), (N, D))
_final_call = _tc(_final_body, (NG, NCLS))


def kernel(x1, x2, edge_index, batch, edge_weight, We1a, be1a, We1b, be1b,
           We2a, be2a, We2b, be2b, Wg1, bg1, Wg2, bg2, Wc1, bc1, Wc2, bc2,
           Whf, bhf, Wh1, bh1, Wh2, bh2):
    row, col = edge_index[0], edge_index[1]
    degp = _deg_call(col, edge_weight)
    (DIS,) = _dis_call(degp)
    t1, u, h2 = _enc_call(x1, x2, We1a, be1a, We1b, be1b, We2a, be2a,
                          We2b, be2b, Wg1, DIS)
    Pa = _spmm_call(t1, row, col, edge_weight)
    Pb = _spmm_call(u, row, col, edge_weight)
    t2, v, c1acc = _mid1_call(Pa, Pb, DIS, bg1, Wg2, h2, Wc1[0], Wc1[1])
    Pc = _spmm_call(t2, row, col, edge_weight)
    Pd = _spmm_call(v, row, col, edge_weight)
    g2, c1, w1 = _mid2_call(Pc, Pd, DIS, bg2, h2, c1acc, Wc1[2], bc1)
    Pe = _spmm_call(w1, row, col, edge_weight)
    vb, c2acc = _mid3_call(Pe, DIS, c1, Wc2[0], Wc2[1])
    Pf = _spmm_call(vb, row, col, edge_weight)
    (logits,) = _final_call(Pf, DIS, c1, c2acc, Wc2[2], bc2, g2,
                            Whf[:D], Whf[D:], bhf, batch, Wh1, bh1, Wh2, bh2)
    return logits


# speedup vs baseline: 1.3123x; 1.3123x over previous
import jax
import jax.numpy as jnp
from jax import lax
from jax.experimental import pallas as pl
from jax.experimental.pallas import tpu as pltpu
from jax.experimental.pallas import tpu_sc as plsc

N = 10000
E = 320000
D = 128
NG = 16
NCLS = 10

SC_CORES = 2
SC_TILES = 16
NW = SC_CORES * SC_TILES
EPW = E // NW
CH = 80
NCHUNK = EPW // CH
ROWS_T = 640
ROWS_LAST = N - (SC_TILES - 1) * ROWS_T

_MESH = plsc.VectorSubcoreMesh(
    core_axis_name="c", subcore_axis_name="s",
    num_cores=SC_CORES, num_subcores=SC_TILES)


def _zero_rows_buf(rows, width):
    z = jnp.zeros((16,), jnp.float32)

    def zr(j, c):
        for k in range(width // 16):
            rows[j, pl.ds(16 * k, 16)] = z
        return c
    lax.fori_loop(0, CH, zr, 0)


def _zero_acc(rows, acc, sid, width):
    _zero_rows_buf(rows, width)
    zsl = rows.at[pl.ds(0, CH)]
    start = pl.multiple_of(sid * ROWS_T, 16)

    @pl.when(sid < SC_TILES - 1)
    def _():
        for k in range(ROWS_T // CH):
            pltpu.sync_copy(zsl, acc.at[pl.ds(start + CH * k, CH)])

    @pl.when(sid == SC_TILES - 1)
    def _():
        for k in range(ROWS_LAST // CH):
            pltpu.sync_copy(zsl, acc.at[pl.ds(start + CH * k, CH)])


def _writeback(acc, out_hbm, cid, sid):
    start = pl.multiple_of(sid * ROWS_T, 16)

    @pl.when(sid < SC_TILES - 1)
    def _():
        pltpu.sync_copy(acc.at[pl.ds(start, ROWS_T)],
                        out_hbm.at[cid, pl.ds(start, ROWS_T)])

    @pl.when(sid == SC_TILES - 1)
    def _():
        pltpu.sync_copy(acc.at[pl.ds(start, ROWS_LAST)],
                        out_hbm.at[cid, pl.ds(start, ROWS_LAST)])


NBUF = 4
ROUNDS = (NCHUNK - 1) // NBUF


_SKIP_SCALE = False


def _scale_rows(rows, wv, b):
    if _SKIP_SCALE:
        return

    def grp(g, c):
        wvec = wv[b, pl.ds(16 * g, 16)]
        for k in range(16):
            wb = jnp.full((16,), wvec[k], jnp.float32)
            j = b * CH + 16 * g + k
            for m in range(D // 16):
                sl = pl.ds(16 * m, 16)
                rows[j, sl] = rows[j, sl] * wb
        return c
    lax.fori_loop(0, CH // 16, grp, 0)


def _fill_rows(rows, wv, b):
    def grp(g, c):
        wvec = wv[b, pl.ds(16 * g, 16)]
        for k in range(16):
            wb = jnp.full((16,), wvec[k], jnp.float32)
            j = b * CH + 16 * g + k
            for m in range(D // 16):
                rows[j, pl.ds(16 * m, 16)] = wb
        return c
    lax.fori_loop(0, CH // 16, grp, 0)


def _deg_body(col_hbm, w_hbm, out_hbm, acc, colv, wv, rows,
              stg_w, stg_c, sc_sem):
    cid = lax.axis_index("c")
    sid = lax.axis_index("s")
    wid = cid * SC_TILES + sid
    base = pl.multiple_of(wid * EPW, 16)

    def off_of(g, b):
        return pl.multiple_of(base + (g * NBUF + b) * CH, 16)

    for b in range(NBUF):
        pltpu.async_copy(w_hbm.at[pl.ds(off_of(0, b), CH)], wv.at[b],
                         stg_w.at[b])
    _zero_acc(rows, acc, sid, D)
    plsc.subcore_barrier()

    def rnd(g, carry):
        for b in range(NBUF):
            @pl.when(g > 0)
            def _():
                pltpu.make_async_copy(rows.at[pl.ds(b * CH, CH)],
                                      acc.at[colv.at[b]], sc_sem.at[b]).wait()
            pltpu.async_copy(col_hbm.at[pl.ds(off_of(g, b), CH)],
                             colv.at[b], stg_c.at[b])
        for b in range(NBUF):
            rsl = rows.at[pl.ds(b * CH, CH)]
            pltpu.make_async_copy(w_hbm.at[pl.ds(0, CH)], wv.at[b],
                                  stg_w.at[b]).wait()
            _fill_rows(rows, wv, b)

            @pl.when(g < ROUNDS - 1)
            def _():
                pltpu.async_copy(w_hbm.at[pl.ds(off_of(g + 1, b), CH)],
                                 wv.at[b], stg_w.at[b])
            pltpu.make_async_copy(col_hbm.at[pl.ds(0, CH)], colv.at[b],
                                  stg_c.at[b]).wait()
            pltpu.async_copy(rsl, acc.at[colv.at[b]], sc_sem.at[b], add=True)
        return carry
    lax.fori_loop(0, ROUNDS, rnd, 0)
    for b in range(NBUF):
        pltpu.make_async_copy(rows.at[pl.ds(b * CH, CH)],
                              acc.at[colv.at[b]], sc_sem.at[b]).wait()
    off = pl.multiple_of(base + (NCHUNK - 1) * CH, 16)
    rsl0 = rows.at[pl.ds(0, CH)]
    pltpu.sync_copy(col_hbm.at[pl.ds(off, CH)], colv.at[0])
    pltpu.sync_copy(w_hbm.at[pl.ds(off, CH)], wv.at[0])
    _fill_rows(rows, wv, 0)
    pltpu.sync_copy(rsl0, acc.at[colv.at[0]], add=True)
    plsc.subcore_barrier()
    _writeback(acc, out_hbm, cid, sid)


def _spmm_body(h_hbm, row_hbm, col_hbm, w_hbm, out_hbm,
               acc, rowv, colv, wv, rows, stg_r, stg_w, stg_c, g_sem, sc_sem):
    cid = lax.axis_index("c")
    sid = lax.axis_index("s")
    wid = cid * SC_TILES + sid
    base = pl.multiple_of(wid * EPW, 16)

    def off_of(g, b):
        return pl.multiple_of(base + (g * NBUF + b) * CH, 16)

    for b in range(NBUF):
        pltpu.async_copy(row_hbm.at[pl.ds(off_of(0, b), CH)], rowv.at[b],
                         stg_r.at[b])
        pltpu.async_copy(w_hbm.at[pl.ds(off_of(0, b), CH)], wv.at[b],
                         stg_w.at[b])
    _zero_acc(rows, acc, sid, D)
    plsc.subcore_barrier()

    def rnd(g, carry):
        for b in range(NBUF):
            rsl = rows.at[pl.ds(b * CH, CH)]

            @pl.when(g > 0)
            def _():
                pltpu.make_async_copy(rsl, acc.at[colv.at[b]],
                                      sc_sem.at[b]).wait()
            pltpu.async_copy(col_hbm.at[pl.ds(off_of(g, b), CH)],
                             colv.at[b], stg_c.at[b])
            pltpu.make_async_copy(row_hbm.at[pl.ds(0, CH)], rowv.at[b],
                                  stg_r.at[b]).wait()
            pltpu.async_copy(h_hbm.at[rowv.at[b]], rsl, g_sem.at[b])
        for b in range(NBUF):
            rsl = rows.at[pl.ds(b * CH, CH)]
            pltpu.make_async_copy(h_hbm.at[rowv.at[b]], rsl,
                                  g_sem.at[b]).wait()

            @pl.when(g < ROUNDS - 1)
            def _():
                pltpu.async_copy(row_hbm.at[pl.ds(off_of(g + 1, b), CH)],
                                 rowv.at[b], stg_r.at[b])
            pltpu.make_async_copy(w_hbm.at[pl.ds(0, CH)], wv.at[b],
                                  stg_w.at[b]).wait()
            _scale_rows(rows, wv, b)

            @pl.when(g < ROUNDS - 1)
            def _():
                pltpu.async_copy(w_hbm.at[pl.ds(off_of(g + 1, b), CH)],
                                 wv.at[b], stg_w.at[b])
            pltpu.make_async_copy(col_hbm.at[pl.ds(0, CH)], colv.at[b],
                                  stg_c.at[b]).wait()
            pltpu.async_copy(rsl, acc.at[colv.at[b]], sc_sem.at[b], add=True)
        return carry
    lax.fori_loop(0, ROUNDS, rnd, 0)
    for b in range(NBUF):
        pltpu.make_async_copy(rows.at[pl.ds(b * CH, CH)],
                              acc.at[colv.at[b]], sc_sem.at[b]).wait()
    off = pl.multiple_of(base + (NCHUNK - 1) * CH, 16)
    rsl0 = rows.at[pl.ds(0, CH)]
    pltpu.sync_copy(row_hbm.at[pl.ds(off, CH)], rowv.at[0])
    pltpu.sync_copy(col_hbm.at[pl.ds(off, CH)], colv.at[0])
    pltpu.sync_copy(w_hbm.at[pl.ds(off, CH)], wv.at[0])
    pltpu.async_copy(h_hbm.at[rowv.at[0]], rsl0, g_sem.at[0]).wait()
    _scale_rows(rows, wv, 0)
    pltpu.sync_copy(rsl0, acc.at[colv.at[0]], add=True)
    plsc.subcore_barrier()
    _writeback(acc, out_hbm, cid, sid)


_deg_call = pl.kernel(
    _deg_body,
    out_type=jax.ShapeDtypeStruct((SC_CORES, N, D), jnp.float32),
    mesh=_MESH,
    scratch_types=[
        pltpu.VMEM_SHARED((N, D), jnp.float32),
        pltpu.VMEM((NBUF, CH), jnp.int32),
        pltpu.VMEM((NBUF, CH), jnp.float32),
        pltpu.VMEM((NBUF * CH, D), jnp.float32),
        pltpu.SemaphoreType.DMA((NBUF,)),
        pltpu.SemaphoreType.DMA((NBUF,)),
        pltpu.SemaphoreType.DMA((NBUF,)),
    ],
)

_spmm_call = pl.kernel(
    _spmm_body,
    out_type=jax.ShapeDtypeStruct((SC_CORES, N, D), jnp.float32),
    mesh=_MESH,
    scratch_types=[
        pltpu.VMEM_SHARED((N, D), jnp.float32),
        pltpu.VMEM((NBUF, CH), jnp.int32),
        pltpu.VMEM((NBUF, CH), jnp.int32),
        pltpu.VMEM((NBUF, CH), jnp.float32),
        pltpu.VMEM((NBUF * CH, D), jnp.float32),
        pltpu.SemaphoreType.DMA((NBUF,)),
        pltpu.SemaphoreType.DMA((NBUF,)),
        pltpu.SemaphoreType.DMA((NBUF,)),
        pltpu.SemaphoreType.DMA((NBUF,)),
        pltpu.SemaphoreType.DMA((NBUF,)),
    ],
)


def _relu(x):
    return jnp.maximum(x, 0.0)


def _dis_body(degp, dis):
    deg = degp[0, :, 0:1] + degp[1, :, 0:1]
    dis1 = jnp.where(deg > 0, lax.rsqrt(jnp.maximum(deg, 1e-12)), 0.0)
    dis[...] = jnp.broadcast_to(dis1, (N, D))


def _enc_body(x1, x2, We1a, be1a, We1b, be1b, We2a, be2a, We2b, be2b,
              Wg1, DIS, t1, u, h2o):
    f32 = jnp.float32
    h1 = _relu(jnp.dot(x1[...], We1a[...], preferred_element_type=f32)
               + be1a[...])
    h1 = jnp.dot(h1, We1b[...], preferred_element_type=f32) + be1b[...]
    h2 = _relu(jnp.dot(x2[...], We2a[...], preferred_element_type=f32)
               + be2a[...])
    h2 = jnp.dot(h2, We2b[...], preferred_element_type=f32) + be2b[...]
    t1[...] = DIS[...] * jnp.dot(h1, Wg1[...], preferred_element_type=f32)
    u[...] = DIS[...] * h2
    h2o[...] = h2


def _mid1_body(Pa, Pb, DIS, bg1, Wg2, h2, Wc1_0, Wc1_1, t2, v, c1acc):
    f32 = jnp.float32
    g = _relu(DIS[...] * (Pa[0] + Pa[1]) + bg1[...])
    t2[...] = DIS[...] * jnp.dot(g, Wg2[...], preferred_element_type=f32)
    Tx1 = -(DIS[...] * (Pb[0] + Pb[1]))
    v[...] = DIS[...] * Tx1
    c1acc[...] = (jnp.dot(h2[...], Wc1_0[...], preferred_element_type=f32)
                  + jnp.dot(Tx1, Wc1_1[...], preferred_element_type=f32))


def _mid2_body(Pc, Pd, DIS, bg2, h2, c1acc, Wc1_2, bc1, g2, c1, w1):
    f32 = jnp.float32
    g2[...] = _relu(DIS[...] * (Pc[0] + Pc[1]) + bg2[...])
    Tx2 = -2.0 * (DIS[...] * (Pd[0] + Pd[1])) - h2[...]
    c1v = _relu(c1acc[...]
                + jnp.dot(Tx2, Wc1_2[...], preferred_element_type=f32)
                + bc1[...])
    c1[...] = c1v
    w1[...] = DIS[...] * c1v


def _mid3_body(Pe, DIS, c1, Wc2_0, Wc2_1, vb, c2acc):
    f32 = jnp.float32
    Tx1b = -(DIS[...] * (Pe[0] + Pe[1]))
    vb[...] = DIS[...] * Tx1b
    c2acc[...] = (jnp.dot(c1[...], Wc2_0[...], preferred_element_type=f32)
                  + jnp.dot(Tx1b, Wc2_1[...], preferred_element_type=f32))


def _final_body(Pf, DIS, c1, c2acc, Wc2_2, bc2, g2, WhfT, WhfB, bhf,
                batch, Wh1, bh1, Wh2, bh2, logits):
    f32 = jnp.float32
    Tx2b = -2.0 * (DIS[...] * (Pf[0] + Pf[1])) - c1[...]
    c2 = _relu(c2acc[...]
               + jnp.dot(Tx2b, Wc2_2[...], preferred_element_type=f32)
               + bc2[...])
    fh = _relu(jnp.dot(g2[...], WhfT[...], preferred_element_type=f32)
               + jnp.dot(c2, WhfB[...], preferred_element_type=f32)
               + bhf[...])
    gid = lax.broadcasted_iota(jnp.int32, (NG, N), 0)
    oh = (gid == jnp.broadcast_to(batch[...], (NG, N))).astype(f32)
    sums = jnp.dot(oh, fh, preferred_element_type=f32)
    cnt = jnp.sum(oh, axis=1, keepdims=True)
    pooled = sums / jnp.maximum(cnt, 1.0)
    hh = _relu(jnp.dot(pooled, Wh1[...], preferred_element_type=f32)
               + bh1[...])
    logits[...] = jnp.dot(hh, Wh2[...], preferred_element_type=f32) + bh2[...]


def _tc(body, *outs):
    return pl.pallas_call(body, out_shape=[jax.ShapeDtypeStruct(s, jnp.float32)
                                           for s in outs])


_dis_call = _tc(_dis_body, (N, D))
_enc_call = _tc(_enc_body, (N, D), (N, D), (N, D))
_mid1_call = _tc(_mid1_body, (N, D), (N, D), (N, D))
_mid2_call = _tc(_mid2_body, (N, D), (N, D), (N, D))
_mid3_call = _tc(_mid3_body, (N, D), (N, D))
_final_call = _tc(_final_body, (NG, NCLS))


def kernel(x1, x2, edge_index, batch, edge_weight, We1a, be1a, We1b, be1b,
           We2a, be2a, We2b, be2b, Wg1, bg1, Wg2, bg2, Wc1, bc1, Wc2, bc2,
           Whf, bhf, Wh1, bh1, Wh2, bh2):
    row, col = edge_index[0], edge_index[1]
    degp = _deg_call(col, edge_weight)
    (DIS,) = _dis_call(degp)
    t1, u, h2 = _enc_call(x1, x2, We1a, be1a, We1b, be1b, We2a, be2a,
                          We2b, be2b, Wg1, DIS)
    Pa = _spmm_call(t1, row, col, edge_weight)
    Pb = _spmm_call(u, row, col, edge_weight)
    t2, v, c1acc = _mid1_call(Pa, Pb, DIS, bg1, Wg2, h2, Wc1[0], Wc1[1])
    Pc = _spmm_call(t2, row, col, edge_weight)
    Pd = _spmm_call(v, row, col, edge_weight)
    g2, c1, w1 = _mid2_call(Pc, Pd, DIS, bg2, h2, c1acc, Wc1[2], bc1)
    Pe = _spmm_call(w1, row, col, edge_weight)
    vb, c2acc = _mid3_call(Pe, DIS, c1, Wc2[0], Wc2[1])
    Pf = _spmm_call(vb, row, col, edge_weight)
    (logits,) = _final_call(Pf, DIS, c1, c2acc, Wc2[2], bc2, g2,
                            Whf[:D], Whf[D:], bhf, batch, Wh1, bh1, Wh2, bh2)
    return logits
